# baseline (speedup 1.0000x reference)
import functools

import jax
import jax.numpy as jnp
import numpy as np
from jax import lax
from jax.experimental import pallas as pl
from jax.experimental.pallas import tpu as pltpu
from jax.experimental.pallas import tpu_sc as plsc

N = 10000
E = 320000
H = 66
HP = 128
QP = 128

NC = 2
NS = 16
NW = NC * NS
EPT = E // NW
C = 80
NCHUNK = EPT // C
TR = 632
N2 = NS * TR

NBUF_M = 2
NGRP_M = NCHUNK // NBUF_M
NBUF_G = 5
NGRP_G = NCHUNK // NBUF_G

_SC_MESH = dict(core_axis_name="c", subcore_axis_name="s",
                num_cores=NC, num_subcores=NS)

_EV = np.array([32 * j + i for j in range(4) for i in range(16)], np.int32)
_OD = _EV + 16


def _pad2(w, r, c):
    return jnp.pad(w, ((0, r - w.shape[0]), (0, c - w.shape[1])))


def _pad1(b, c):
    return jnp.pad(b, (0, c - b.shape[0])).reshape(1, c)


def _t_h_body(x_ref, w_ref, b_ref, o_ref):
    o_ref[...] = jnp.dot(x_ref[...], w_ref[...],
                         preferred_element_type=jnp.float32) + b_ref[...]


def _t_ea_body(a_ref, w_ref, b_ref, o_ref):
    a = a_ref[...]
    w = w_ref[...]
    b = b_ref[...]
    o_ref[...] = jnp.concatenate(
        [jnp.dot(a[:, :16], w, preferred_element_type=jnp.float32) + b,
         jnp.dot(a[:, 16:], w, preferred_element_type=jnp.float32) + b],
        axis=1)


def _t_node_body(h_ref, ag_ref, w1_ref, b1_ref, w2_ref, b2_ref,
                 g_ref, be_ref, wa_ref, wb_ref, hn_ref, hs_ref, hd_ref):
    h = h_ref[...]
    ag = ag_ref[...]
    u = h + (ag[0] + ag[1])[:N]
    z = jnp.dot(jax.nn.relu(jnp.dot(u, w1_ref[...],
                                    preferred_element_type=jnp.float32)
                            + b1_ref[...]),
                w2_ref[...], preferred_element_type=jnp.float32) + b2_ref[...]
    m = jnp.mean(z, axis=0, keepdims=True)
    v = jnp.mean((z - m) ** 2, axis=0, keepdims=True)
    zn = (z - m) * lax.rsqrt(v + 1e-5) * g_ref[...] + be_ref[...]
    hn = (h + jax.nn.relu(zn)) * 0.5
    hn_ref[...] = hn
    hs_ref[...] = jnp.dot(hn, wa_ref[...], preferred_element_type=jnp.float32)
    hd_ref[...] = jnp.dot(hn, wb_ref[...], preferred_element_type=jnp.float32)


def _unpack_pairs(p32):
    lo = lax.bitcast_convert_type(p32 << 16, jnp.float32)
    hi = lax.bitcast_convert_type(p32 & jnp.int32(-65536), jnp.float32)
    return lo, hi


def _pair_mm(ea_l, ea_r, w_ref, b2_ref):
    return jnp.concatenate(
        [jnp.dot(ea_l, w_ref[...], preferred_element_type=jnp.float32),
         jnp.dot(ea_r, w_ref[...], preferred_element_type=jnp.float32)],
        axis=1) + b2_ref[...]


def _t_edge_body(g_ref, ea_ref, w1e_ref, w1o_ref, b1e2_ref, b1o2_ref,
                 w2e_ref, w2o_ref, b2_ref, o_ref):
    ea = ea_ref[...]
    ea_l = ea[:, :HP]
    ea_r = ea[:, HP:]
    glo, ghi = _unpack_pairs(g_ref[...])
    te = jax.nn.relu(glo + _pair_mm(ea_l, ea_r, w1e_ref, b1e2_ref))
    to = jax.nn.relu(ghi + _pair_mm(ea_l, ea_r, w1o_ref, b1o2_ref))
    d_l = (jnp.dot(te[:, :64], w2e_ref[...],
                   preferred_element_type=jnp.float32)
           + jnp.dot(to[:, :64], w2o_ref[...],
                     preferred_element_type=jnp.float32) + b2_ref[...])
    d_r = (jnp.dot(te[:, 64:], w2e_ref[...],
                   preferred_element_type=jnp.float32)
           + jnp.dot(to[:, 64:], w2o_ref[...],
                     preferred_element_type=jnp.float32) + b2_ref[...])
    o_ref[...] = ea + jnp.concatenate([d_l, d_r], axis=1) * 0.5


def _t_q_body(h_ref, wa_ref, wb_ref, q1_ref, q2_ref):
    hr = jax.nn.relu(h_ref[...])
    q1_ref[...] = jnp.dot(hr, wa_ref[...], preferred_element_type=jnp.float32)
    q2_ref[...] = jnp.dot(hr, wb_ref[...], preferred_element_type=jnp.float32)


def _t_final_body(gq_ref, ea_ref, w1e_ref, w1o_ref, b1e2_ref, b1o2_ref,
                  w2e_ref, w2o_ref, b2_ref, w3_ref, b3_ref, o_ref):
    ea = ea_ref[...]
    ea_l = ea[:, :HP]
    ea_r = ea[:, HP:]
    qlo, qhi = _unpack_pairs(gq_ref[...])
    oe = jax.nn.relu(qlo + _pair_mm(ea_l, ea_r, w1e_ref, b1e2_ref))
    oo = jax.nn.relu(qhi + _pair_mm(ea_l, ea_r, w1o_ref, b1o2_ref))

    def head(par):
        sl = slice(0, 64) if par == 0 else slice(64, 128)
        o2 = jax.nn.relu(jnp.dot(oe[:, sl], w2e_ref[...],
                                 preferred_element_type=jnp.float32)
                         + jnp.dot(oo[:, sl], w2o_ref[...],
                                   preferred_element_type=jnp.float32)
                         + b2_ref[...])
        return jnp.dot(o2, w3_ref[...],
                       preferred_element_type=jnp.float32) + b3_ref[...]

    o_ref[...] = jnp.concatenate([head(0), head(1)], axis=1)


BL = 2560


def _edge_spec(d):
    return pl.BlockSpec((BL, d), lambda i: (i, 0))


def _w_spec(r, c):
    return pl.BlockSpec((r, c), lambda i: (0, 0))


def _sc_msg_body(h_h, ea_h, src_h, dst_h, out_h,
                 srcb, dstb, rows0, rows1, ea0, ea1, zbuf_v, aggr_sh,
                 semI, semG, semE, semS, semZ):
    c = lax.axis_index("c")
    s = lax.axis_index("s")
    wid = c * NS + s
    base0 = wid * EPT
    rows = [rows0, rows1]
    eab = [ea0, ea1]

    def _zf(r, _):
        for j in range(HP // 16):
            zbuf_v[r, pl.ds(j * 16, 16)] = jnp.zeros((16,), jnp.float32)
        return 0
    lax.fori_loop(0, 8, _zf, 0)

    def _zc(k, _):
        pltpu.async_copy(zbuf_v, aggr_sh.at[pl.ds(s * TR + k * 8, 8)], semZ)
        return 0
    lax.fori_loop(0, TR // 8, _zc, 0)

    def _zw(k, _):
        pltpu.make_async_copy(zbuf_v, aggr_sh.at[pl.ds(s * TR + k * 8, 8)],
                              semZ).wait()
        return 0
    lax.fori_loop(0, TR // 8, _zw, 0)
    plsc.subcore_barrier()

    def _do_chunks(k0, nb):
        di, de = [], []
        for b in range(nb):
            base = base0 + (k0 + b) * C
            di.append(pltpu.async_copy(src_h.at[pl.ds(base, C)],
                                       srcb.at[b], semI))
            di.append(pltpu.async_copy(dst_h.at[pl.ds(base, C)],
                                       dstb.at[b], semI))
            de.append(pltpu.async_copy(
                ea_h.at[pl.ds(pl.multiple_of(base // 2, 8), C // 2)],
                eab[b], semE))
        for d_ in di:
            d_.wait()
        dg = [pltpu.async_copy(h_h.at[srcb.at[b]], rows[b], semG)
              for b in range(nb)]
        ds_ = []
        for b in range(nb):
            dg[b].wait()
            de[b].wait()

            def _relu_add(p, _, b=b):
                for rr in range(2):
                    r = p * 2 + rr
                    for j in range(HP // 16):
                        sl = pl.ds(j * 16, 16)
                        sp = pl.ds(HP * rr + j * 16, 16)
                        rows[b][r, sl] = jnp.maximum(
                            rows[b][r, sl] + eab[b][p, sp], 0.0)
                return 0
            lax.fori_loop(0, C // 2, _relu_add, 0)
            ds_.append(pltpu.async_copy(rows[b], aggr_sh.at[dstb.at[b]],
                                        semS, add=True))
        for d_ in ds_:
            d_.wait()

    def _group(g, _):
        _do_chunks(g * NBUF_M, NBUF_M)
        return 0
    lax.fori_loop(0, NGRP_M, _group, 0)
    _do_chunks(NGRP_M * NBUF_M, NCHUNK - NGRP_M * NBUF_M)

    plsc.subcore_barrier()
    pltpu.sync_copy(aggr_sh.at[pl.ds(s * TR, TR)],
                    out_h.at[c, pl.ds(s * TR, TR)])


@functools.lru_cache(maxsize=None)
def _make_sc_msg():
    return functools.partial(
        pl.kernel,
        out_type=jax.ShapeDtypeStruct((NC, N2, HP), jnp.float32),
        mesh=plsc.VectorSubcoreMesh(**_SC_MESH),
        scratch_types=[
        pltpu.VMEM((NBUF_M, C), jnp.int32),
        pltpu.VMEM((NBUF_M, C), jnp.int32),
        pltpu.VMEM((C, HP), jnp.float32),
        pltpu.VMEM((C, HP), jnp.float32),
        pltpu.VMEM((C // 2, 2 * HP), jnp.float32),
        pltpu.VMEM((C // 2, 2 * HP), jnp.float32),
        pltpu.VMEM((8, HP), jnp.float32),
        pltpu.VMEM_SHARED((N2, HP), jnp.float32),
            pltpu.SemaphoreType.DMA,
            pltpu.SemaphoreType.DMA,
            pltpu.SemaphoreType.DMA,
            pltpu.SemaphoreType.DMA,
            pltpu.SemaphoreType.DMA,
        ],
    )(_sc_msg_body)


def _sc_gather_sum_body(d, ta_h, tb_h, src2_h, dst2_h, out_h,
                        srcA, dstA, b0, b1, b2, b3, b4,
                        o0, o1, o2, o3, o4, semG, semO):
    c = lax.axis_index("c")
    s = lax.axis_index("s")
    wid = c * NS + s
    base0 = wid * EPT
    bufs = [b0, b1, b2, b3, b4]
    obufs = [o0, o1, o2, o3, o4]

    pltpu.sync_copy(src2_h.at[wid], srcA)
    pltpu.sync_copy(dst2_h.at[wid], dstA)

    def _group(g, _):
        k0 = g * NBUF_G
        d1 = [pltpu.async_copy(ta_h.at[srcA.at[k0 + b]], bufs[b], semG)
              for b in range(NBUF_G)]
        for d_ in d1:
            d_.wait()
        d2 = [pltpu.async_copy(tb_h.at[dstA.at[k0 + b]], bufs[b], semG,
                               add=True)
              for b in range(NBUF_G)]
        d3 = []
        for b in range(NBUF_G):
            d2[b].wait()

            def _pack(p, _, b=b):
                for rr in range(2):
                    r = p * 2 + rr
                    for j in range(d // 32):
                        pa = lax.bitcast_convert_type(
                            bufs[b][r, pl.ds(32 * j, 16)], jnp.int32)
                        pb = lax.bitcast_convert_type(
                            bufs[b][r, pl.ds(32 * j + 16, 16)], jnp.int32)
                        v = (lax.shift_right_logical(pa + 0x8000, 16)
                             | ((pb + 0x8000) & jnp.int32(-65536)))
                        obufs[b][p, pl.ds(64 * rr + 16 * j, 16)] = v
                return 0
            lax.fori_loop(0, C // 2, _pack, 0)
            d3.append(pltpu.async_copy(
                obufs[b],
                out_h.at[pl.ds(pl.multiple_of(
                    (base0 + (k0 + b) * C) // 2, 8), C // 2)], semO))
        for d_ in d3:
            d_.wait()
        return 0
    lax.fori_loop(0, NGRP_G, _group, 0)


@functools.lru_cache(maxsize=None)
def _make_sc_gather_sum(d):
    return functools.partial(
        pl.kernel,
        out_type=jax.ShapeDtypeStruct((E // 2, d), jnp.int32),
        mesh=plsc.VectorSubcoreMesh(**_SC_MESH),
        scratch_types=[
            pltpu.VMEM((NCHUNK, C), jnp.int32),
            pltpu.VMEM((NCHUNK, C), jnp.int32),
            pltpu.VMEM((C, d), jnp.float32),
            pltpu.VMEM((C, d), jnp.float32),
            pltpu.VMEM((C, d), jnp.float32),
            pltpu.VMEM((C, d), jnp.float32),
            pltpu.VMEM((C, d), jnp.float32),
            pltpu.VMEM((C // 2, d), jnp.int32),
            pltpu.VMEM((C // 2, d), jnp.int32),
            pltpu.VMEM((C // 2, d), jnp.int32),
            pltpu.VMEM((C // 2, d), jnp.int32),
            pltpu.VMEM((C // 2, d), jnp.int32),
            pltpu.SemaphoreType.DMA,
            pltpu.SemaphoreType.DMA,
        ],
    )(functools.partial(_sc_gather_sum_body, d))


def kernel(x, edge_index, edge_attr, edge_label_index, target_edge_attr, params):
    del edge_label_index, target_edge_attr
    P = params
    src = edge_index[0]
    dst = edge_index[1]
    src2 = src.reshape(NW, NCHUNK, C)
    dst2 = dst.reshape(NW, NCHUNK, C)

    node_W = _pad2(P["node_W"], 128, HP)
    node_b = _pad1(P["node_b"], HP)
    edge_W = _pad2(P["edge_W"], 16, HP)
    edge_b = _pad1(P["edge_b"], HP)

    h = pl.pallas_call(
        _t_h_body,
        out_shape=jax.ShapeDtypeStruct((N, HP), jnp.float32),
    )(x, node_W, node_b)

    BL2 = BL // 2
    g_spec = pl.BlockSpec((BL2, HP), lambda i: (i, 0))
    ea_spec = pl.BlockSpec((BL2, 2 * HP), lambda i: (i, 0))

    ea = pl.pallas_call(
        _t_ea_body,
        grid=(E // BL,),
        in_specs=[pl.BlockSpec((BL2, 32), lambda i: (i, 0)),
                  _w_spec(16, HP), _w_spec(1, HP)],
        out_specs=ea_spec,
        out_shape=jax.ShapeDtypeStruct((E // 2, 2 * HP), jnp.float32),
    )(edge_attr.reshape(E // 2, 32), edge_W, edge_b)

    for lp in P["layers"]:
        cW1 = _pad2(lp["cW1"], HP, HP)
        cb1 = _pad1(lp["cb1"], HP)
        cW2 = _pad2(lp["cW2"], HP, HP)
        cb2 = _pad1(lp["cb2"], HP)
        gam = _pad1(lp["gamma"], HP)
        bet = _pad1(lp["beta"], HP)
        eW1a = _pad2(lp["eW1"][:H], HP, HP)
        eW1b = _pad2(lp["eW1"][H:2 * H], HP, HP)
        eW1c = _pad2(lp["eW1"][2 * H:], HP, HP)
        eb1 = _pad1(lp["eb1"], HP)
        eW2 = _pad2(lp["eW2"], HP, HP)
        eb2 = _pad1(lp["eb2"], HP)

        aggr2 = _make_sc_msg()(h, ea, src, dst)

        h, hs1, hd1 = pl.pallas_call(
            _t_node_body,
            out_shape=[jax.ShapeDtypeStruct((N, HP), jnp.float32)] * 3,
        )(h, aggr2, cW1, cb1, cW2, cb2, gam, bet, eW1a, eW1b)

        g = _make_sc_gather_sum(HP)(hs1, hd1, src2, dst2)

        ea = pl.pallas_call(
            _t_edge_body,
            grid=(E // BL,),
            in_specs=[g_spec, ea_spec,
                      _w_spec(HP, 64), _w_spec(HP, 64),
                      _w_spec(1, HP), _w_spec(1, HP),
                      _w_spec(64, HP), _w_spec(64, HP), _w_spec(1, HP)],
            out_specs=ea_spec,
            out_shape=jax.ShapeDtypeStruct((E // 2, 2 * HP), jnp.float32),
        )(g, ea, eW1c[:, _EV], eW1c[:, _OD],
          jnp.concatenate([eb1[:, _EV]] * 2, axis=1),
          jnp.concatenate([eb1[:, _OD]] * 2, axis=1),
          eW2[_EV, :], eW2[_OD, :], eb2)

    mW1a = _pad2(P["mW1"][:H], HP, QP)
    mW1b = _pad2(P["mW1"][H:2 * H], HP, QP)
    mW1c = _pad2(P["mW1"][2 * H:], HP, QP)
    mb1 = _pad1(P["mb1"], QP)
    mW2 = _pad2(P["mW2"], QP, 32)
    mb2 = _pad1(P["mb2"], 32)
    mW3 = _pad2(P["mW3"], 32, 2)
    mb3 = P["mb3"].reshape(1, 2)

    q1, q2 = pl.pallas_call(
        _t_q_body,
        out_shape=[jax.ShapeDtypeStruct((N, QP), jnp.float32)] * 2,
    )(h, mW1a, mW1b)

    gq = _make_sc_gather_sum(QP)(q1, q2, src2, dst2)

    out = pl.pallas_call(
        _t_final_body,
        grid=(E // BL,),
        in_specs=[g_spec, ea_spec,
                  _w_spec(HP, 64), _w_spec(HP, 64),
                  _w_spec(1, HP), _w_spec(1, HP),
                  _w_spec(64, 32), _w_spec(64, 32), _w_spec(1, 32),
                  _w_spec(32, 2), _w_spec(1, 2)],
        out_specs=pl.BlockSpec((BL2, 4), lambda i: (i, 0)),
        out_shape=jax.ShapeDtypeStruct((E // 2, 4), jnp.float32),
    )(gq, ea, mW1c[:, _EV], mW1c[:, _OD],
      jnp.concatenate([mb1[:, _EV]] * 2, axis=1),
      jnp.concatenate([mb1[:, _OD]] * 2, axis=1),
      mW2[_EV, :], mW2[_OD, :], mb2, mW3, mb3)

    return out.reshape(E, 2)


# speedup vs baseline: 1.5586x; 1.5586x over previous
import functools

import jax
import jax.numpy as jnp
from jax import lax
from jax.experimental import pallas as pl
from jax.experimental.pallas import tpu as pltpu
from jax.experimental.pallas import tpu_sc as plsc

N = 10000
E = 320000
H = 66
HP = 128
QP = 128

NC = 2
NS = 16
NW = NC * NS
EPT = E // NW
C = 80
NCHUNK = EPT // C
TR = 632
N2 = NS * TR

NBUF_M = 2
NGRP_M = NCHUNK // NBUF_M
NBUF_G = 5
NGRP_G = NCHUNK // NBUF_G

_SC_MESH = dict(core_axis_name="c", subcore_axis_name="s",
                num_cores=NC, num_subcores=NS)


def _pad2(w, r, c):
    return jnp.pad(w, ((0, r - w.shape[0]), (0, c - w.shape[1])))


def _pad1(b, c):
    return jnp.pad(b, (0, c - b.shape[0])).reshape(1, c)


def _t_h_body(x_ref, w_ref, b_ref, o_ref):
    o_ref[...] = jnp.dot(x_ref[...], w_ref[...],
                         preferred_element_type=jnp.float32) + b_ref[...]


def _t_ea_body(a_ref, w_ref, b_ref, o_ref):
    o_ref[...] = jnp.dot(a_ref[...], w_ref[...],
                         preferred_element_type=jnp.float32) + b_ref[...]


def _t_node_body(h_ref, ag_ref, w1_ref, b1_ref, w2_ref, b2_ref, g_ref, be_ref,
                 wa_ref, wb_ref, hn_ref, hs_ref, hd_ref):
    h = h_ref[...]
    ag = ag_ref[...]
    u = h + (ag[0] + ag[1])[:N]
    z = jnp.dot(jax.nn.relu(jnp.dot(u, w1_ref[...],
                                    preferred_element_type=jnp.float32)
                            + b1_ref[...]),
                w2_ref[...], preferred_element_type=jnp.float32) + b2_ref[...]
    m = jnp.mean(z, axis=0, keepdims=True)
    v = jnp.mean((z - m) ** 2, axis=0, keepdims=True)
    zn = (z - m) * lax.rsqrt(v + 1e-5) * g_ref[...] + be_ref[...]
    hn = (h + jax.nn.relu(zn)) * 0.5
    hn_ref[...] = hn
    hs_ref[...] = jnp.dot(hn, wa_ref[...], preferred_element_type=jnp.float32)
    hd_ref[...] = jnp.dot(hn, wb_ref[...], preferred_element_type=jnp.float32)


def _t_edge_body(g_ref, ea_ref, w1_ref, b1_ref, w2_ref, b2_ref, o_ref):
    ea = ea_ref[...]
    t = jax.nn.relu(g_ref[...]
                    + jnp.dot(ea, w1_ref[...],
                              preferred_element_type=jnp.float32)
                    + b1_ref[...])
    o_ref[...] = ea + (jnp.dot(t, w2_ref[...],
                               preferred_element_type=jnp.float32)
                       + b2_ref[...]) * 0.5


def _t_q_body(h_ref, wa_ref, wb_ref, q1_ref, q2_ref):
    hr = jax.nn.relu(h_ref[...])
    q1_ref[...] = jnp.dot(hr, wa_ref[...], preferred_element_type=jnp.float32)
    q2_ref[...] = jnp.dot(hr, wb_ref[...], preferred_element_type=jnp.float32)


def _t_final_body(gq_ref, ea_ref, w1_ref, b1_ref, w2_ref, b2_ref,
                  w3_ref, b3_ref, o_ref):
    o = jax.nn.relu(gq_ref[...]
                    + jnp.dot(ea_ref[...], w1_ref[...],
                              preferred_element_type=jnp.float32)
                    + b1_ref[...])
    o = jax.nn.relu(jnp.dot(o, w2_ref[...],
                            preferred_element_type=jnp.float32) + b2_ref[...])
    o_ref[...] = jnp.dot(o, w3_ref[...],
                         preferred_element_type=jnp.float32) + b3_ref[...]


BL = 2560


def _edge_spec(d):
    return pl.BlockSpec((BL, d), lambda i: (i, 0))


def _w_spec(r, c):
    return pl.BlockSpec((r, c), lambda i: (0, 0))


def _sc_msg_body(h_h, ea_h, src_h, dst_h, out_h,
                 srcb, dstb, rows0, rows1, ea0, ea1, zbuf_v, aggr_sh,
                 semI, semG, semE, semS, semZ):
    c = lax.axis_index("c")
    s = lax.axis_index("s")
    wid = c * NS + s
    base0 = wid * EPT
    rows = [rows0, rows1]
    eab = [ea0, ea1]

    def _zf(r, _):
        for j in range(HP // 16):
            zbuf_v[r, pl.ds(j * 16, 16)] = jnp.zeros((16,), jnp.float32)
        return 0
    lax.fori_loop(0, 8, _zf, 0)

    def _zc(k, _):
        pltpu.async_copy(zbuf_v, aggr_sh.at[pl.ds(s * TR + k * 8, 8)], semZ)
        return 0
    lax.fori_loop(0, TR // 8, _zc, 0)

    def _zw(k, _):
        pltpu.make_async_copy(zbuf_v, aggr_sh.at[pl.ds(s * TR + k * 8, 8)],
                              semZ).wait()
        return 0
    lax.fori_loop(0, TR // 8, _zw, 0)
    plsc.subcore_barrier()

    def _do_chunks(k0, nb):
        di, de = [], []
        for b in range(nb):
            base = base0 + (k0 + b) * C
            di.append(pltpu.async_copy(src_h.at[pl.ds(base, C)],
                                       srcb.at[b], semI))
            di.append(pltpu.async_copy(dst_h.at[pl.ds(base, C)],
                                       dstb.at[b], semI))
            de.append(pltpu.async_copy(ea_h.at[pl.ds(base, C)],
                                       eab[b], semE))
        for d_ in di:
            d_.wait()
        dg = [pltpu.async_copy(h_h.at[srcb.at[b]], rows[b], semG)
              for b in range(nb)]
        ds_ = []
        for b in range(nb):
            dg[b].wait()
            de[b].wait()

            def _relu_add(r, _, b=b):
                for j in range(HP // 16):
                    sl = pl.ds(j * 16, 16)
                    rows[b][r, sl] = jnp.maximum(
                        rows[b][r, sl] + eab[b][r, sl], 0.0)
                return 0
            lax.fori_loop(0, C, _relu_add, 0)
            ds_.append(pltpu.async_copy(rows[b], aggr_sh.at[dstb.at[b]],
                                        semS, add=True))
        for d_ in ds_:
            d_.wait()

    def _group(g, _):
        _do_chunks(g * NBUF_M, NBUF_M)
        return 0
    lax.fori_loop(0, NGRP_M, _group, 0)
    _do_chunks(NGRP_M * NBUF_M, NCHUNK - NGRP_M * NBUF_M)

    plsc.subcore_barrier()
    pltpu.sync_copy(aggr_sh.at[pl.ds(s * TR, TR)],
                    out_h.at[c, pl.ds(s * TR, TR)])


@functools.lru_cache(maxsize=None)
def _make_sc_msg():
    return functools.partial(
        pl.kernel,
        out_type=jax.ShapeDtypeStruct((NC, N2, HP), jnp.float32),
        mesh=plsc.VectorSubcoreMesh(**_SC_MESH),
        scratch_types=[
            pltpu.VMEM((NBUF_M, C), jnp.int32),
            pltpu.VMEM((NBUF_M, C), jnp.int32),
            pltpu.VMEM((C, HP), jnp.float32),
            pltpu.VMEM((C, HP), jnp.float32),
            pltpu.VMEM((C, HP), jnp.float32),
            pltpu.VMEM((C, HP), jnp.float32),
            pltpu.VMEM((8, HP), jnp.float32),
            pltpu.VMEM_SHARED((N2, HP), jnp.float32),
            pltpu.SemaphoreType.DMA,
            pltpu.SemaphoreType.DMA,
            pltpu.SemaphoreType.DMA,
            pltpu.SemaphoreType.DMA,
            pltpu.SemaphoreType.DMA,
        ],
    )(_sc_msg_body)


def _sc_gather_sum_body(d, ta_h, tb_h, src2_h, dst2_h, out_h,
                        srcA, dstA, b0, b1, b2, b3, b4, semG, semO):
    c = lax.axis_index("c")
    s = lax.axis_index("s")
    wid = c * NS + s
    base0 = wid * EPT
    bufs = [b0, b1, b2, b3, b4]

    pltpu.sync_copy(src2_h.at[wid], srcA)
    pltpu.sync_copy(dst2_h.at[wid], dstA)

    def _group(g, _):
        k0 = g * NBUF_G
        d1 = [pltpu.async_copy(ta_h.at[srcA.at[k0 + b]], bufs[b], semG)
              for b in range(NBUF_G)]
        for d_ in d1:
            d_.wait()
        d2 = [pltpu.async_copy(tb_h.at[dstA.at[k0 + b]], bufs[b], semG,
                               add=True)
              for b in range(NBUF_G)]
        for d_ in d2:
            d_.wait()
        d3 = [pltpu.async_copy(bufs[b],
                               out_h.at[pl.ds(base0 + (k0 + b) * C, C)],
                               semO)
              for b in range(NBUF_G)]
        for d_ in d3:
            d_.wait()
        return 0
    lax.fori_loop(0, NGRP_G, _group, 0)


@functools.lru_cache(maxsize=None)
def _make_sc_gather_sum(d):
    return functools.partial(
        pl.kernel,
        out_type=jax.ShapeDtypeStruct((E, d), jnp.float32),
        mesh=plsc.VectorSubcoreMesh(**_SC_MESH),
        scratch_types=[
            pltpu.VMEM((NCHUNK, C), jnp.int32),
            pltpu.VMEM((NCHUNK, C), jnp.int32),
            pltpu.VMEM((C, d), jnp.float32),
            pltpu.VMEM((C, d), jnp.float32),
            pltpu.VMEM((C, d), jnp.float32),
            pltpu.VMEM((C, d), jnp.float32),
            pltpu.VMEM((C, d), jnp.float32),
            pltpu.SemaphoreType.DMA,
            pltpu.SemaphoreType.DMA,
        ],
    )(functools.partial(_sc_gather_sum_body, d))


def kernel(x, edge_index, edge_attr, edge_label_index, target_edge_attr, params):
    del edge_label_index, target_edge_attr
    P = params
    src = edge_index[0]
    dst = edge_index[1]
    src2 = src.reshape(NW, NCHUNK, C)
    dst2 = dst.reshape(NW, NCHUNK, C)

    node_W = _pad2(P["node_W"], 128, HP)
    node_b = _pad1(P["node_b"], HP)
    edge_W = _pad2(P["edge_W"], 16, HP)
    edge_b = _pad1(P["edge_b"], HP)

    h = pl.pallas_call(
        _t_h_body,
        out_shape=jax.ShapeDtypeStruct((N, HP), jnp.float32),
    )(x, node_W, node_b)

    ea = pl.pallas_call(
        _t_ea_body,
        grid=(E // BL,),
        in_specs=[_edge_spec(16), _w_spec(16, HP), _w_spec(1, HP)],
        out_specs=_edge_spec(HP),
        out_shape=jax.ShapeDtypeStruct((E, HP), jnp.float32),
    )(edge_attr, edge_W, edge_b)

    for lp in P["layers"]:
        cW1 = _pad2(lp["cW1"], HP, HP)
        cb1 = _pad1(lp["cb1"], HP)
        cW2 = _pad2(lp["cW2"], HP, HP)
        cb2 = _pad1(lp["cb2"], HP)
        gam = _pad1(lp["gamma"], HP)
        bet = _pad1(lp["beta"], HP)
        eW1a = _pad2(lp["eW1"][:H], HP, HP)
        eW1b = _pad2(lp["eW1"][H:2 * H], HP, HP)
        eW1c = _pad2(lp["eW1"][2 * H:], HP, HP)
        eb1 = _pad1(lp["eb1"], HP)
        eW2 = _pad2(lp["eW2"], HP, HP)
        eb2 = _pad1(lp["eb2"], HP)

        aggr2 = _make_sc_msg()(h, ea, src, dst)

        h, hs1, hd1 = pl.pallas_call(
            _t_node_body,
            out_shape=[jax.ShapeDtypeStruct((N, HP), jnp.float32)] * 3,
        )(h, aggr2, cW1, cb1, cW2, cb2, gam, bet, eW1a, eW1b)

        g = _make_sc_gather_sum(HP)(hs1, hd1, src2, dst2)

        ea = pl.pallas_call(
            _t_edge_body,
            grid=(E // BL,),
            in_specs=[_edge_spec(HP), _edge_spec(HP), _w_spec(HP, HP),
                      _w_spec(1, HP), _w_spec(HP, HP), _w_spec(1, HP)],
            out_specs=_edge_spec(HP),
            out_shape=jax.ShapeDtypeStruct((E, HP), jnp.float32),
        )(g, ea, eW1c, eb1, eW2, eb2)

    mW1a = _pad2(P["mW1"][:H], HP, QP)
    mW1b = _pad2(P["mW1"][H:2 * H], HP, QP)
    mW1c = _pad2(P["mW1"][2 * H:], HP, QP)
    mb1 = _pad1(P["mb1"], QP)
    mW2 = _pad2(P["mW2"], QP, 32)
    mb2 = _pad1(P["mb2"], 32)
    mW3 = _pad2(P["mW3"], 32, 2)
    mb3 = P["mb3"].reshape(1, 2)

    q1, q2 = pl.pallas_call(
        _t_q_body,
        out_shape=[jax.ShapeDtypeStruct((N, QP), jnp.float32)] * 2,
    )(h, mW1a, mW1b)

    gq = _make_sc_gather_sum(QP)(q1, q2, src2, dst2)

    out = pl.pallas_call(
        _t_final_body,
        grid=(E // BL,),
        in_specs=[_edge_spec(QP), _edge_spec(HP), _w_spec(HP, QP),
                  _w_spec(1, QP), _w_spec(QP, 32), _w_spec(1, 32),
                  _w_spec(32, 2), _w_spec(1, 2)],
        out_specs=_edge_spec(2),
        out_shape=jax.ShapeDtypeStruct((E, 2), jnp.float32),
    )(gq, ea, mW1c, mb1, mW2, mb2, mW3, mb3)

    return out


# speedup vs baseline: 1.5845x; 1.0166x over previous
import functools

import jax
import jax.numpy as jnp
from jax import lax
from jax.experimental import pallas as pl
from jax.experimental.pallas import tpu as pltpu
from jax.experimental.pallas import tpu_sc as plsc

N = 10000
E = 320000
H = 66
HP = 128
QP = 128

NC = 2
NS = 16
NW = NC * NS
EPT = E // NW
C = 80
NCHUNK = EPT // C
TR = 632
N2 = NS * TR

NBUF_M = 2
NGRP_M = NCHUNK // NBUF_M
NBUF_G = 5
NGRP_G = NCHUNK // NBUF_G

_SC_MESH = dict(core_axis_name="c", subcore_axis_name="s",
                num_cores=NC, num_subcores=NS)


def _pad2(w, r, c):
    return jnp.pad(w, ((0, r - w.shape[0]), (0, c - w.shape[1])))


def _pad1(b, c):
    return jnp.pad(b, (0, c - b.shape[0])).reshape(1, c)


def _t_h_body(x_ref, w_ref, b_ref, o_ref):
    o_ref[...] = jnp.dot(x_ref[...], w_ref[...],
                         preferred_element_type=jnp.float32) + b_ref[...]


def _t_ea_body(a_ref, w_ref, b_ref, o_ref):
    o_ref[...] = jnp.dot(a_ref[...], w_ref[...],
                         preferred_element_type=jnp.float32) + b_ref[...]


def _t_node_body(h_ref, ag_ref, w1_ref, b1_ref, w2_ref, b2_ref, g_ref, be_ref,
                 wa_ref, wb_ref, hn_ref, hs_ref, hd_ref):
    h = h_ref[...]
    ag = ag_ref[...]
    u = h + (ag[0] + ag[1])[:N]
    z = jnp.dot(jax.nn.relu(jnp.dot(u, w1_ref[...],
                                    preferred_element_type=jnp.float32)
                            + b1_ref[...]),
                w2_ref[...], preferred_element_type=jnp.float32) + b2_ref[...]
    m = jnp.mean(z, axis=0, keepdims=True)
    v = jnp.mean((z - m) ** 2, axis=0, keepdims=True)
    zn = (z - m) * lax.rsqrt(v + 1e-5) * g_ref[...] + be_ref[...]
    hn = (h + jax.nn.relu(zn)) * 0.5
    hn_ref[...] = hn
    hs_ref[...] = jnp.dot(hn, wa_ref[...], preferred_element_type=jnp.float32)
    hd_ref[...] = jnp.dot(hn, wb_ref[...], preferred_element_type=jnp.float32)


def _t_edge_body(g_ref, ea_ref, w1_ref, b1_ref, w2_ref, b2_ref, o_ref):
    ea = ea_ref[...]
    t = jax.nn.relu(g_ref[...]
                    + jnp.dot(ea, w1_ref[...],
                              preferred_element_type=jnp.float32)
                    + b1_ref[...])
    o_ref[...] = ea + (jnp.dot(t, w2_ref[...],
                               preferred_element_type=jnp.float32)
                       + b2_ref[...]) * 0.5


def _t_q_body(h_ref, wa_ref, wb_ref, q1_ref, q2_ref):
    hr = jax.nn.relu(h_ref[...])
    q1_ref[...] = jnp.dot(hr, wa_ref[...], preferred_element_type=jnp.float32)
    q2_ref[...] = jnp.dot(hr, wb_ref[...], preferred_element_type=jnp.float32)


def _t_final_body(gq_ref, ea_ref, w1_ref, b1_ref, w2_ref, b2_ref,
                  w3_ref, b3_ref, o_ref):
    o = jax.nn.relu(gq_ref[...]
                    + jnp.dot(ea_ref[...], w1_ref[...],
                              preferred_element_type=jnp.float32)
                    + b1_ref[...])
    o = jax.nn.relu(jnp.dot(o, w2_ref[...],
                            preferred_element_type=jnp.float32) + b2_ref[...])
    o_ref[...] = jnp.dot(o, w3_ref[...],
                         preferred_element_type=jnp.float32) + b3_ref[...]


BL = 2560


def _edge_spec(d):
    return pl.BlockSpec((BL, d), lambda i: (i, 0))


def _w_spec(r, c):
    return pl.BlockSpec((r, c), lambda i: (0, 0))


def _sc_msg_body(h_h, ea_h, src_h, dst_h, out_h,
                 srcb, dstb, rows0, rows1, ea0, ea1, zbuf_v, aggr_sh,
                 semI, semG, semE, semS, semZ):
    c = lax.axis_index("c")
    s = lax.axis_index("s")
    wid = c * NS + s
    base0 = wid * EPT
    rows = [rows0, rows1]
    eab = [ea0, ea1]

    def _zf(r, _):
        for j in range(HP // 16):
            zbuf_v[r, pl.ds(j * 16, 16)] = jnp.zeros((16,), jnp.float32)
        return 0
    lax.fori_loop(0, 8, _zf, 0)

    def _zc(k, _):
        pltpu.async_copy(zbuf_v, aggr_sh.at[pl.ds(s * TR + k * 8, 8)], semZ)
        return 0
    lax.fori_loop(0, TR // 8, _zc, 0)

    def _zw(k, _):
        pltpu.make_async_copy(zbuf_v, aggr_sh.at[pl.ds(s * TR + k * 8, 8)],
                              semZ).wait()
        return 0
    lax.fori_loop(0, TR // 8, _zw, 0)
    plsc.subcore_barrier()

    def _do_chunks(k0, nb):
        di, de = [], []
        for b in range(nb):
            base = base0 + (k0 + b) * C
            di.append(pltpu.async_copy(src_h.at[pl.ds(base, C)],
                                       srcb.at[b], semI))
            di.append(pltpu.async_copy(dst_h.at[pl.ds(base, C)],
                                       dstb.at[b], semI))
            de.append(pltpu.async_copy(ea_h.at[pl.ds(base, C)],
                                       eab[b], semE))
        for d_ in di:
            d_.wait()
        dg = [pltpu.async_copy(h_h.at[srcb.at[b]], rows[b], semG)
              for b in range(nb)]
        ds_ = []
        for b in range(nb):
            dg[b].wait()
            de[b].wait()

            def _relu_add(r, _, b=b):
                for j in range(HP // 16):
                    sl = pl.ds(j * 16, 16)
                    rows[b][r, sl] = jnp.maximum(
                        rows[b][r, sl] + eab[b][r, sl], 0.0)
                return 0
            lax.fori_loop(0, C, _relu_add, 0)
            ds_.append(pltpu.async_copy(rows[b], aggr_sh.at[dstb.at[b]],
                                        semS, add=True))
        for d_ in ds_:
            d_.wait()

    def _group(g, _):
        _do_chunks(g * NBUF_M, NBUF_M)
        return 0
    lax.fori_loop(0, NGRP_M, _group, 0)
    _do_chunks(NGRP_M * NBUF_M, NCHUNK - NGRP_M * NBUF_M)

    plsc.subcore_barrier()
    pltpu.sync_copy(aggr_sh.at[pl.ds(s * TR, TR)],
                    out_h.at[c, pl.ds(s * TR, TR)])


@functools.lru_cache(maxsize=None)
def _make_sc_msg():
    return functools.partial(
        pl.kernel,
        out_type=jax.ShapeDtypeStruct((NC, N2, HP), jnp.float32),
        mesh=plsc.VectorSubcoreMesh(**_SC_MESH),
        scratch_types=[
            pltpu.VMEM((NBUF_M, C), jnp.int32),
            pltpu.VMEM((NBUF_M, C), jnp.int32),
            pltpu.VMEM((C, HP), jnp.float32),
            pltpu.VMEM((C, HP), jnp.float32),
            pltpu.VMEM((C, HP), jnp.float32),
            pltpu.VMEM((C, HP), jnp.float32),
            pltpu.VMEM((8, HP), jnp.float32),
            pltpu.VMEM_SHARED((N2, HP), jnp.float32),
            pltpu.SemaphoreType.DMA,
            pltpu.SemaphoreType.DMA,
            pltpu.SemaphoreType.DMA,
            pltpu.SemaphoreType.DMA,
            pltpu.SemaphoreType.DMA,
        ],
    )(_sc_msg_body)


def _sc_gather_sum_body(d, ta_h, tb_h, src2_h, dst2_h, out_h,
                        srcA, dstA, big, semG, semO):
    c = lax.axis_index("c")
    s = lax.axis_index("s")
    wid = c * NS + s
    base0 = wid * EPT
    bufs = [big.at[pl.ds(b * C, C)] for b in range(NBUF_G)]

    pltpu.sync_copy(src2_h.at[wid], srcA)
    pltpu.sync_copy(dst2_h.at[wid], dstA)

    def _group(g, _):
        k0 = g * NBUF_G
        d1 = [pltpu.async_copy(ta_h.at[srcA.at[k0 + b]], bufs[b], semG)
              for b in range(NBUF_G)]
        d2 = []
        for b in range(NBUF_G):
            d1[b].wait()
            d2.append(pltpu.async_copy(tb_h.at[dstA.at[k0 + b]], bufs[b],
                                       semG, add=True))
        for d_ in d2:
            d_.wait()
        pltpu.async_copy(big, out_h.at[pl.ds(base0 + k0 * C, NBUF_G * C)],
                         semO).wait()
        return 0
    lax.fori_loop(0, NGRP_G, _group, 0)


@functools.lru_cache(maxsize=None)
def _make_sc_gather_sum(d):
    return functools.partial(
        pl.kernel,
        out_type=jax.ShapeDtypeStruct((E, d), jnp.float32),
        mesh=plsc.VectorSubcoreMesh(**_SC_MESH),
        scratch_types=[
            pltpu.VMEM((NCHUNK, C), jnp.int32),
            pltpu.VMEM((NCHUNK, C), jnp.int32),
            pltpu.VMEM((NBUF_G * C, d), jnp.float32),
            pltpu.SemaphoreType.DMA,
            pltpu.SemaphoreType.DMA,
        ],
    )(functools.partial(_sc_gather_sum_body, d))


def kernel(x, edge_index, edge_attr, edge_label_index, target_edge_attr, params):
    del edge_label_index, target_edge_attr
    P = params
    src = edge_index[0]
    dst = edge_index[1]
    src2 = src.reshape(NW, NCHUNK, C)
    dst2 = dst.reshape(NW, NCHUNK, C)

    node_W = _pad2(P["node_W"], 128, HP)
    node_b = _pad1(P["node_b"], HP)
    edge_W = _pad2(P["edge_W"], 16, HP)
    edge_b = _pad1(P["edge_b"], HP)

    h = pl.pallas_call(
        _t_h_body,
        out_shape=jax.ShapeDtypeStruct((N, HP), jnp.float32),
    )(x, node_W, node_b)

    ea = pl.pallas_call(
        _t_ea_body,
        grid=(E // BL,),
        in_specs=[_edge_spec(16), _w_spec(16, HP), _w_spec(1, HP)],
        out_specs=_edge_spec(HP),
        out_shape=jax.ShapeDtypeStruct((E, HP), jnp.float32),
    )(edge_attr, edge_W, edge_b)

    for lp in P["layers"]:
        cW1 = _pad2(lp["cW1"], HP, HP)
        cb1 = _pad1(lp["cb1"], HP)
        cW2 = _pad2(lp["cW2"], HP, HP)
        cb2 = _pad1(lp["cb2"], HP)
        gam = _pad1(lp["gamma"], HP)
        bet = _pad1(lp["beta"], HP)
        eW1a = _pad2(lp["eW1"][:H], HP, HP)
        eW1b = _pad2(lp["eW1"][H:2 * H], HP, HP)
        eW1c = _pad2(lp["eW1"][2 * H:], HP, HP)
        eb1 = _pad1(lp["eb1"], HP)
        eW2 = _pad2(lp["eW2"], HP, HP)
        eb2 = _pad1(lp["eb2"], HP)

        aggr2 = _make_sc_msg()(h, ea, src, dst)

        h, hs1, hd1 = pl.pallas_call(
            _t_node_body,
            out_shape=[jax.ShapeDtypeStruct((N, HP), jnp.float32)] * 3,
        )(h, aggr2, cW1, cb1, cW2, cb2, gam, bet, eW1a, eW1b)

        g = _make_sc_gather_sum(HP)(hs1, hd1, src2, dst2)

        ea = pl.pallas_call(
            _t_edge_body,
            grid=(E // BL,),
            in_specs=[_edge_spec(HP), _edge_spec(HP), _w_spec(HP, HP),
                      _w_spec(1, HP), _w_spec(HP, HP), _w_spec(1, HP)],
            out_specs=_edge_spec(HP),
            out_shape=jax.ShapeDtypeStruct((E, HP), jnp.float32),
        )(g, ea, eW1c, eb1, eW2, eb2)

    mW1a = _pad2(P["mW1"][:H], HP, QP)
    mW1b = _pad2(P["mW1"][H:2 * H], HP, QP)
    mW1c = _pad2(P["mW1"][2 * H:], HP, QP)
    mb1 = _pad1(P["mb1"], QP)
    mW2 = _pad2(P["mW2"], QP, 32)
    mb2 = _pad1(P["mb2"], 32)
    mW3 = _pad2(P["mW3"], 32, 2)
    mb3 = P["mb3"].reshape(1, 2)

    q1, q2 = pl.pallas_call(
        _t_q_body,
        out_shape=[jax.ShapeDtypeStruct((N, QP), jnp.float32)] * 2,
    )(h, mW1a, mW1b)

    gq = _make_sc_gather_sum(QP)(q1, q2, src2, dst2)

    out = pl.pallas_call(
        _t_final_body,
        grid=(E // BL,),
        in_specs=[_edge_spec(QP), _edge_spec(HP), _w_spec(HP, QP),
                  _w_spec(1, QP), _w_spec(QP, 32), _w_spec(1, 32),
                  _w_spec(32, 2), _w_spec(1, 2)],
        out_specs=_edge_spec(2),
        out_shape=jax.ShapeDtypeStruct((E, 2), jnp.float32),
    )(gq, ea, mW1c, mb1, mW2, mb2, mW3, mb3)

    return out


# speedup vs baseline: 1.6285x; 1.0278x over previous
import functools

import jax
import jax.numpy as jnp
from jax import lax
from jax.experimental import pallas as pl
from jax.experimental.pallas import tpu as pltpu
from jax.experimental.pallas import tpu_sc as plsc

N = 10000
E = 320000
H = 66
HP = 128
QP = 128

NC = 2
NS = 16
NW = NC * NS
EPT = E // NW
C = 80
NCHUNK = EPT // C
TR = 632
N2 = NS * TR

NBUF_M = 2
NGRP_M = NCHUNK // NBUF_M
NBUF_G = 5
NGRP_G = NCHUNK // NBUF_G

_SC_MESH = dict(core_axis_name="c", subcore_axis_name="s",
                num_cores=NC, num_subcores=NS)


def _pad2(w, r, c):
    return jnp.pad(w, ((0, r - w.shape[0]), (0, c - w.shape[1])))


def _pad1(b, c):
    return jnp.pad(b, (0, c - b.shape[0])).reshape(1, c)


def _t_h_body(x_ref, w_ref, b_ref, o_ref):
    o_ref[...] = jnp.dot(x_ref[...], w_ref[...],
                         preferred_element_type=jnp.float32) + b_ref[...]


def _t_ea_body(a_ref, w_ref, b_ref, o_ref):
    o_ref[...] = jnp.dot(a_ref[...], w_ref[...],
                         preferred_element_type=jnp.float32) + b_ref[...]


def _t_node_body(h_ref, ag_ref, w1_ref, b1_ref, w2_ref, b2_ref, g_ref, be_ref,
                 wa_ref, wb_ref, hn_ref, hs_ref, hd_ref):
    h = h_ref[...]
    ag = ag_ref[...]
    u = h + (ag[0] + ag[1])[:N]
    z = jnp.dot(jax.nn.relu(jnp.dot(u, w1_ref[...],
                                    preferred_element_type=jnp.float32)
                            + b1_ref[...]),
                w2_ref[...], preferred_element_type=jnp.float32) + b2_ref[...]
    m = jnp.mean(z, axis=0, keepdims=True)
    v = jnp.mean((z - m) ** 2, axis=0, keepdims=True)
    zn = (z - m) * lax.rsqrt(v + 1e-5) * g_ref[...] + be_ref[...]
    hn = (h + jax.nn.relu(zn)) * 0.5
    hn_ref[...] = hn
    hs_ref[...] = jnp.dot(hn, wa_ref[...], preferred_element_type=jnp.float32)
    hd_ref[...] = jnp.dot(hn, wb_ref[...], preferred_element_type=jnp.float32)


def _t_edge_body(g_ref, ea_ref, w1_ref, b1_ref, w2_ref, b2_ref, o_ref):
    ea = ea_ref[...]
    t = jax.nn.relu(g_ref[...]
                    + jnp.dot(ea, w1_ref[...],
                              preferred_element_type=jnp.float32)
                    + b1_ref[...])
    o_ref[...] = ea + (jnp.dot(t, w2_ref[...],
                               preferred_element_type=jnp.float32)
                       + b2_ref[...]) * 0.5


def _t_q_body(h_ref, wa_ref, wb_ref, q1_ref, q2_ref):
    hr = jax.nn.relu(h_ref[...])
    q1_ref[...] = jnp.dot(hr, wa_ref[...], preferred_element_type=jnp.float32)
    q2_ref[...] = jnp.dot(hr, wb_ref[...], preferred_element_type=jnp.float32)


def _t_final_body(gq_ref, ea_ref, w1_ref, b1_ref, w2_ref, b2_ref,
                  w3_ref, b3_ref, o_ref):
    o = jax.nn.relu(gq_ref[...]
                    + jnp.dot(ea_ref[...], w1_ref[...],
                              preferred_element_type=jnp.float32)
                    + b1_ref[...])
    o = jax.nn.relu(jnp.dot(o, w2_ref[...],
                            preferred_element_type=jnp.float32) + b2_ref[...])
    o_ref[...] = jnp.dot(o, w3_ref[...],
                         preferred_element_type=jnp.float32) + b3_ref[...]


BL = 2560


def _edge_spec(d):
    return pl.BlockSpec((BL, d), lambda i: (i, 0))


def _w_spec(r, c):
    return pl.BlockSpec((r, c), lambda i: (0, 0))


def _sc_msg_body(h_h, ea_h, src_h, dst_h, out_h,
                 srcb, dstb, rows0, rows1, ea0, ea1, zbuf_v, aggr_sh,
                 semI, semG, semE, semS, semZ):
    c = lax.axis_index("c")
    s = lax.axis_index("s")
    wid = c * NS + s
    base0 = wid * EPT
    rows = [rows0, rows1]
    eab = [ea0, ea1]

    def _zf(r, _):
        for j in range(HP // 16):
            zbuf_v[r, pl.ds(j * 16, 16)] = jnp.zeros((16,), jnp.float32)
        return 0
    lax.fori_loop(0, 8, _zf, 0)

    def _zc(k, _):
        pltpu.async_copy(zbuf_v, aggr_sh.at[pl.ds(s * TR + k * 8, 8)], semZ)
        return 0
    lax.fori_loop(0, TR // 8, _zc, 0)

    def _zw(k, _):
        pltpu.make_async_copy(zbuf_v, aggr_sh.at[pl.ds(s * TR + k * 8, 8)],
                              semZ).wait()
        return 0
    lax.fori_loop(0, TR // 8, _zw, 0)
    plsc.subcore_barrier()

    def _do_chunks(k0, nb):
        di, de = [], []
        for b in range(nb):
            base = base0 + (k0 + b) * C
            di.append(pltpu.async_copy(src_h.at[pl.ds(base, C)],
                                       srcb.at[b], semI))
            di.append(pltpu.async_copy(dst_h.at[pl.ds(base, C)],
                                       dstb.at[b], semI))
            de.append(pltpu.async_copy(ea_h.at[pl.ds(base, C)],
                                       eab[b], semE))
        for d_ in di:
            d_.wait()
        dg = [pltpu.async_copy(h_h.at[srcb.at[b]], rows[b], semG)
              for b in range(nb)]
        ds_ = []
        for b in range(nb):
            dg[b].wait()
            de[b].wait()

            def _relu_add(r, _, b=b):
                for j in range(80 // 16):
                    sl = pl.ds(j * 16, 16)
                    rows[b][r, sl] = jnp.maximum(
                        rows[b][r, sl] + eab[b][r, sl], 0.0)
                return 0
            lax.fori_loop(0, C, _relu_add, 0)
            ds_.append(pltpu.async_copy(rows[b], aggr_sh.at[dstb.at[b]],
                                        semS, add=True))
        for d_ in ds_:
            d_.wait()

    def _group(g, _):
        _do_chunks(g * NBUF_M, NBUF_M)
        return 0
    lax.fori_loop(0, NGRP_M, _group, 0)
    _do_chunks(NGRP_M * NBUF_M, NCHUNK - NGRP_M * NBUF_M)

    plsc.subcore_barrier()
    pltpu.sync_copy(aggr_sh.at[pl.ds(s * TR, TR)],
                    out_h.at[c, pl.ds(s * TR, TR)])


@functools.lru_cache(maxsize=None)
def _make_sc_msg():
    return functools.partial(
        pl.kernel,
        out_type=jax.ShapeDtypeStruct((NC, N2, HP), jnp.float32),
        mesh=plsc.VectorSubcoreMesh(**_SC_MESH),
        scratch_types=[
            pltpu.VMEM((NBUF_M, C), jnp.int32),
            pltpu.VMEM((NBUF_M, C), jnp.int32),
            pltpu.VMEM((C, HP), jnp.float32),
            pltpu.VMEM((C, HP), jnp.float32),
            pltpu.VMEM((C, HP), jnp.float32),
            pltpu.VMEM((C, HP), jnp.float32),
            pltpu.VMEM((8, HP), jnp.float32),
            pltpu.VMEM_SHARED((N2, HP), jnp.float32),
            pltpu.SemaphoreType.DMA,
            pltpu.SemaphoreType.DMA,
            pltpu.SemaphoreType.DMA,
            pltpu.SemaphoreType.DMA,
            pltpu.SemaphoreType.DMA,
        ],
    )(_sc_msg_body)


def _sc_gather_sum_body(d, ta_h, tb_h, src2_h, dst2_h, out_h,
                        srcA, dstA, big, semG, semO):
    c = lax.axis_index("c")
    s = lax.axis_index("s")
    wid = c * NS + s
    base0 = wid * EPT
    bufs = [big.at[pl.ds(b * C, C)] for b in range(NBUF_G)]

    pltpu.sync_copy(src2_h.at[wid], srcA)
    pltpu.sync_copy(dst2_h.at[wid], dstA)

    def _group(g, _):
        k0 = g * NBUF_G
        d1 = [pltpu.async_copy(ta_h.at[srcA.at[k0 + b]], bufs[b], semG)
              for b in range(NBUF_G)]
        d2 = []
        for b in range(NBUF_G):
            d1[b].wait()
            d2.append(pltpu.async_copy(tb_h.at[dstA.at[k0 + b]], bufs[b],
                                       semG, add=True))
        for d_ in d2:
            d_.wait()
        pltpu.async_copy(big, out_h.at[pl.ds(base0 + k0 * C, NBUF_G * C)],
                         semO).wait()
        return 0
    lax.fori_loop(0, NGRP_G, _group, 0)


@functools.lru_cache(maxsize=None)
def _make_sc_gather_sum(d):
    return functools.partial(
        pl.kernel,
        out_type=jax.ShapeDtypeStruct((E, d), jnp.float32),
        mesh=plsc.VectorSubcoreMesh(**_SC_MESH),
        scratch_types=[
            pltpu.VMEM((NCHUNK, C), jnp.int32),
            pltpu.VMEM((NCHUNK, C), jnp.int32),
            pltpu.VMEM((NBUF_G * C, d), jnp.float32),
            pltpu.SemaphoreType.DMA,
            pltpu.SemaphoreType.DMA,
        ],
    )(functools.partial(_sc_gather_sum_body, d))


def kernel(x, edge_index, edge_attr, edge_label_index, target_edge_attr, params):
    del edge_label_index, target_edge_attr
    P = params
    src = edge_index[0]
    dst = edge_index[1]
    src2 = src.reshape(NW, NCHUNK, C)
    dst2 = dst.reshape(NW, NCHUNK, C)

    node_W = _pad2(P["node_W"], 128, HP)
    node_b = _pad1(P["node_b"], HP)
    edge_W = _pad2(P["edge_W"], 16, HP)
    edge_b = _pad1(P["edge_b"], HP)

    h = pl.pallas_call(
        _t_h_body,
        out_shape=jax.ShapeDtypeStruct((N, HP), jnp.float32),
    )(x, node_W, node_b)

    ea = pl.pallas_call(
        _t_ea_body,
        grid=(E // BL,),
        in_specs=[_edge_spec(16), _w_spec(16, HP), _w_spec(1, HP)],
        out_specs=_edge_spec(HP),
        out_shape=jax.ShapeDtypeStruct((E, HP), jnp.float32),
    )(edge_attr, edge_W, edge_b)

    for lp in P["layers"]:
        cW1 = _pad2(lp["cW1"], HP, HP)
        cb1 = _pad1(lp["cb1"], HP)
        cW2 = _pad2(lp["cW2"], HP, HP)
        cb2 = _pad1(lp["cb2"], HP)
        gam = _pad1(lp["gamma"], HP)
        bet = _pad1(lp["beta"], HP)
        eW1a = _pad2(lp["eW1"][:H], HP, HP)
        eW1b = _pad2(lp["eW1"][H:2 * H], HP, HP)
        eW1c = _pad2(lp["eW1"][2 * H:], HP, HP)
        eb1 = _pad1(lp["eb1"], HP)
        eW2 = _pad2(lp["eW2"], HP, HP)
        eb2 = _pad1(lp["eb2"], HP)

        aggr2 = _make_sc_msg()(h, ea, src, dst)

        h, hs1, hd1 = pl.pallas_call(
            _t_node_body,
            out_shape=[jax.ShapeDtypeStruct((N, HP), jnp.float32)] * 3,
        )(h, aggr2, cW1, cb1, cW2, cb2, gam, bet, eW1a, eW1b)

        g = _make_sc_gather_sum(HP)(hs1, hd1, src2, dst2)

        ea = pl.pallas_call(
            _t_edge_body,
            grid=(E // BL,),
            in_specs=[_edge_spec(HP), _edge_spec(HP), _w_spec(HP, HP),
                      _w_spec(1, HP), _w_spec(HP, HP), _w_spec(1, HP)],
            out_specs=_edge_spec(HP),
            out_shape=jax.ShapeDtypeStruct((E, HP), jnp.float32),
        )(g, ea, eW1c, eb1, eW2, eb2)

    mW1a = _pad2(P["mW1"][:H], HP, QP)
    mW1b = _pad2(P["mW1"][H:2 * H], HP, QP)
    mW1c = _pad2(P["mW1"][2 * H:], HP, QP)
    mb1 = _pad1(P["mb1"], QP)
    mW2 = _pad2(P["mW2"], QP, 32)
    mb2 = _pad1(P["mb2"], 32)
    mW3 = _pad2(P["mW3"], 32, 2)
    mb3 = P["mb3"].reshape(1, 2)

    q1, q2 = pl.pallas_call(
        _t_q_body,
        out_shape=[jax.ShapeDtypeStruct((N, QP), jnp.float32)] * 2,
    )(h, mW1a, mW1b)

    gq = _make_sc_gather_sum(QP)(q1, q2, src2, dst2)

    out = pl.pallas_call(
        _t_final_body,
        grid=(E // BL,),
        in_specs=[_edge_spec(QP), _edge_spec(HP), _w_spec(HP, QP),
                  _w_spec(1, QP), _w_spec(QP, 32), _w_spec(1, 32),
                  _w_spec(32, 2), _w_spec(1, 2)],
        out_specs=_edge_spec(2),
        out_shape=jax.ShapeDtypeStruct((E, 2), jnp.float32),
    )(gq, ea, mW1c, mb1, mW2, mb2, mW3, mb3)

    return out
